# baseline (speedup 1.0000x reference)
import jax
import jax.numpy as jnp
from jax import lax
from jax.experimental import pallas as pl
from jax.experimental.pallas import tpu as pltpu
from jax.experimental.pallas import tpu_sc as plsc

NUM_CORES = 2
NUM_SUBCORES = 16
LANES = 16
NUM_WORKERS = NUM_CORES * NUM_SUBCORES

BATCH = 16384
EMBED_DIM = 32
B_PER_W = BATCH // NUM_WORKERS
IDX_CHUNK = 128
NUM_CHUNKS = B_PER_W // IDX_CHUNK
NUM_BLOCKS = B_PER_W // LANES


def _sc_kernel(user_ids_hbm, item_ids_hbm, user_table_hbm, item_table_hbm,
               out_hbm, uidx_v, iidx_v, urows_v, irows_v, out_v, sem):
    wid = lax.axis_index("s") * NUM_CORES + lax.axis_index("c")
    base = wid * B_PER_W

    pltpu.sync_copy(user_ids_hbm.at[pl.ds(base, B_PER_W)], uidx_v)
    pltpu.sync_copy(item_ids_hbm.at[pl.ds(base, B_PER_W)], iidx_v)

    copies = []
    for c in range(NUM_CHUNKS):
        sl = pl.ds(c * IDX_CHUNK, IDX_CHUNK)
        copies.append(pltpu.async_copy(
            user_table_hbm.at[uidx_v.at[sl]], urows_v.at[sl], sem))
        copies.append(pltpu.async_copy(
            item_table_hbm.at[iidx_v.at[sl]], irows_v.at[sl], sem))
    for cp in copies:
        cp.wait()

    lanes = lax.iota(jnp.int32, LANES)

    def block_body(blk, carry):
        rows = blk * LANES + lanes
        acc = jnp.zeros((LANES,), jnp.float32)
        for d in range(EMBED_DIM):
            dvec = jnp.full((LANES,), d, jnp.int32)
            gu = plsc.load_gather(urows_v, [rows, dvec])
            gi = plsc.load_gather(irows_v, [rows, dvec])
            acc = acc + gu * gi
        out_v[pl.ds(blk * LANES, LANES)] = acc
        return carry

    lax.fori_loop(0, NUM_BLOCKS, block_body, 0)

    pltpu.sync_copy(out_v, out_hbm.at[pl.ds(base, B_PER_W)])


@jax.jit
def kernel(user_ids, item_ids, user_table, item_table):
    mesh = plsc.VectorSubcoreMesh(
        core_axis_name="c", subcore_axis_name="s",
        num_cores=NUM_CORES, num_subcores=NUM_SUBCORES)
    run = pl.kernel(
        _sc_kernel,
        out_type=jax.ShapeDtypeStruct((BATCH,), jnp.float32),
        mesh=mesh,
        scratch_types=[
            pltpu.VMEM((B_PER_W,), jnp.int32),
            pltpu.VMEM((B_PER_W,), jnp.int32),
            pltpu.VMEM((B_PER_W, EMBED_DIM), jnp.float32),
            pltpu.VMEM((B_PER_W, EMBED_DIM), jnp.float32),
            pltpu.VMEM((B_PER_W,), jnp.float32),
            pltpu.SemaphoreType.DMA,
        ],
        compiler_params=pltpu.CompilerParams(
            needs_layout_passes=False, use_tc_tiling_on_sc=False),
    )
    return run(user_ids.astype(jnp.int32), item_ids.astype(jnp.int32),
               user_table, item_table)


# speedup vs baseline: 3.7355x; 3.7355x over previous
import jax
import jax.numpy as jnp
from jax import lax
from jax.experimental import pallas as pl
from jax.experimental.pallas import tpu as pltpu
from jax.experimental.pallas import tpu_sc as plsc

NUM_CORES = 2
NUM_SUBCORES = 16
LANES = 16
NUM_WORKERS = NUM_CORES * NUM_SUBCORES

BATCH = 16384
EMBED_DIM = 32
TILE_W = 128
B_PER_W = BATCH // NUM_WORKERS
CHUNK = 16
NUM_CHUNKS = B_PER_W // CHUNK


def _sc_kernel(user_ids_hbm, item_ids_hbm, ut_hbm, it_hbm,
               out_hbm, idxu_v, idxi_v, wbuf, uval, out_v, sem):
    wid = lax.axis_index("s") * NUM_CORES + lax.axis_index("c")
    base = wid * B_PER_W

    pltpu.sync_copy(user_ids_hbm.at[pl.ds(base, B_PER_W)], idxu_v)
    pltpu.sync_copy(item_ids_hbm.at[pl.ds(base, B_PER_W)], idxi_v)

    lanes16 = lax.iota(jnp.int32, LANES)
    row_of_lane = lanes16 * EMBED_DIM

    def fetch_windows(tbl_hbm, bases):
        copies = []
        for q in range(CHUNK):
            col = pl.multiple_of((bases[q] >> 7) << 7, TILE_W)
            copies.append(pltpu.async_copy(
                tbl_hbm.at[:, pl.ds(col, TILE_W)],
                wbuf.at[pl.ds(q * EMBED_DIM, EMBED_DIM), :], sem))
        for cp in copies:
            cp.wait()

    def chunk_body(cb, carry):
        c0 = cb * CHUNK
        uvec = idxu_v[pl.ds(c0, CHUNK)]
        fetch_windows(ut_hbm, uvec)
        lane_u = uvec & (TILE_W - 1)
        for d in range(EMBED_DIM):
            uval[d, :] = plsc.load_gather(wbuf, [row_of_lane + d, lane_u])

        ivec = idxi_v[pl.ds(c0, CHUNK)]
        fetch_windows(it_hbm, ivec)
        lane_i = ivec & (TILE_W - 1)
        acc = jnp.zeros((LANES,), jnp.float32)
        for d in range(EMBED_DIM):
            gi = plsc.load_gather(wbuf, [row_of_lane + d, lane_i])
            acc = acc + uval[d, :] * gi
        out_v[pl.ds(c0, CHUNK)] = acc
        return carry

    lax.fori_loop(0, NUM_CHUNKS, chunk_body, 0)

    pltpu.sync_copy(out_v, out_hbm.at[pl.ds(base, B_PER_W)])


@jax.jit
def kernel(user_ids, item_ids, user_table, item_table):
    mesh = plsc.VectorSubcoreMesh(
        core_axis_name="c", subcore_axis_name="s",
        num_cores=NUM_CORES, num_subcores=NUM_SUBCORES)
    run = pl.kernel(
        _sc_kernel,
        out_type=jax.ShapeDtypeStruct((BATCH,), jnp.float32),
        mesh=mesh,
        scratch_types=[
            pltpu.VMEM((B_PER_W,), jnp.int32),
            pltpu.VMEM((B_PER_W,), jnp.int32),
            pltpu.VMEM((CHUNK * EMBED_DIM, TILE_W), jnp.float32),
            pltpu.VMEM((EMBED_DIM, LANES), jnp.float32),
            pltpu.VMEM((B_PER_W,), jnp.float32),
            pltpu.SemaphoreType.DMA,
        ],
        compiler_params=pltpu.CompilerParams(needs_layout_passes=False),
    )
    return run(user_ids.astype(jnp.int32), item_ids.astype(jnp.int32),
               user_table.T, item_table.T)
